# Initial kernel scaffold; baseline (speedup 1.0000x reference)
import jax
import jax.numpy as jnp
from jax.experimental import pallas as pl


def kernel(x, W1, g1, b1, W2, g2, b2, W3, g3, b3, W4, g4, b4, W5, g5, b5, Wmu, bmu, Wlv, blv, Wd1, bd1, Wd2, bd2, Wp, bp, eps):
    raise NotImplementedError("write your pallas kernel here")


# speedup vs baseline: 10.4930x; 10.4930x over previous
import functools

import jax
import jax.numpy as jnp
from jax import lax
from jax.experimental import pallas as pl
from jax.experimental.pallas import tpu as pltpu
from jax.experimental.pallas import tpu_sc as plsc

_B, _N, _K = 8, 1024, 20
_BN = _B * _N
_KPAD = 32
_NEG = -3.0e38

_NC, _NS, _L = 2, 16, 16
_NW = _NC * _NS
_PTW = _BN // _NW
_GP = 4
_GB = 8
_GRP = _PTW // _GB
_IDXW = _PTW * _K


def _layer_call(cin, cout, first, args):
    def body(*refs):
        if first:
            (xt_ref, waT_ref, wdT_ref, idx_ref, u_ref, w_ref) = refs
            xt = xt_ref[...]
        else:
            (m_ref, mn_ref, wp_ref, sc_ref, sh_ref, waT_ref, wdT_ref,
             xt_out_ref, idx_ref, u_ref, w_ref) = refs
            scale = sc_ref[...]
            shift = sh_ref[...]
            mt = jnp.where(scale >= 0.0, m_ref[0], mn_ref[0])
            y = scale * (mt + wp_ref[...]) + shift
            xt = jnp.where(y > 0.0, y, 0.2 * y)
            xt_out_ref[...] = xt
        b = pl.program_id(0)
        xb = xt.astype(jnp.bfloat16)
        G = lax.dot_general(xb, xb, (((1,), (1,)), ((), ())),
                            preferred_element_type=jnp.float32)
        inner = -2.0 * G
        xsq = xt * xt
        xxcol = jnp.sum(xsq, axis=1, keepdims=True)
        ones_row = jnp.ones((1, cin), jnp.float32)
        xxrow = lax.dot_general(ones_row, xsq, (((1,), (1,)), ((), ())),
                                preferred_element_type=jnp.float32,
                                precision=lax.Precision.HIGHEST)
        pair = ((0.0 - xxcol) - inner) - xxrow
        col = lax.broadcasted_iota(jnp.int32, (_N, _N), 1)
        cols = []
        for k in range(_K):
            m = jnp.max(pair, axis=1, keepdims=True)
            cand = jnp.where(pair == m, col, _N)
            ik = jnp.min(cand, axis=1, keepdims=True)
            cols.append(ik)
            if k + 1 < _K:
                pair = jnp.where(col == ik, _NEG, pair)
        idxmat = jnp.concatenate(cols + [cols[-1]] * (_KPAD - _K), axis=1)
        idx_ref[...] = idxmat + b * _N
        u_ref[...] = lax.dot_general(xt, waT_ref[...], (((1,), (0,)), ((), ())),
                                     preferred_element_type=jnp.float32,
                            precision=lax.Precision.HIGHEST)
        w_ref[...] = lax.dot_general(xt, wdT_ref[...], (((1,), (0,)), ((), ())),
                                     preferred_element_type=jnp.float32,
                            precision=lax.Precision.HIGHEST)

    wspec = [
        pl.BlockSpec((cin, cout), lambda b: (0, 0)),
        pl.BlockSpec((cin, cout), lambda b: (0, 0)),
    ]
    if first:
        in_specs = [pl.BlockSpec((_N, cin), lambda b: (b, 0))] + wspec
        out_shapes = []
        out_specs = []
    else:
        in_specs = [
            pl.BlockSpec((1, _N, cin), lambda b: (0, b, 0)),
            pl.BlockSpec((1, _N, cin), lambda b: (1, b, 0)),
            pl.BlockSpec((_N, cin), lambda b: (b, 0)),
            pl.BlockSpec((1, cin), lambda b: (0, 0)),
            pl.BlockSpec((1, cin), lambda b: (0, 0)),
        ] + wspec
        out_shapes = [jax.ShapeDtypeStruct((_BN, cin), jnp.float32)]
        out_specs = [pl.BlockSpec((_N, cin), lambda b: (b, 0))]
    out_shapes += [
        jax.ShapeDtypeStruct((_BN, _KPAD), jnp.int32),
        jax.ShapeDtypeStruct((_BN, cout), jnp.float32),
        jax.ShapeDtypeStruct((_BN, cout), jnp.float32),
    ]
    out_specs += [
        pl.BlockSpec((_N, _KPAD), lambda b: (b, 0)),
        pl.BlockSpec((_N, cout), lambda b: (b, 0)),
        pl.BlockSpec((_N, cout), lambda b: (b, 0)),
    ]
    return pl.pallas_call(
        body,
        grid=(_B,),
        in_specs=in_specs,
        out_specs=out_specs,
        out_shape=out_shapes,
    )(*args)


def _sc_gather(cout):
    mesh = plsc.VectorSubcoreMesh(core_axis_name="c", subcore_axis_name="s",
                                  num_cores=_NC, num_subcores=_NS)

    @functools.partial(
        pl.kernel,
        out_type=jax.ShapeDtypeStruct((4, _BN, cout), jnp.float32),
        mesh=mesh,
        scratch_types=[
            pltpu.VMEM((_IDXW,), jnp.int32),
            pltpu.VMEM((_GP * _K, cout), jnp.float32),
            pltpu.VMEM((4, _GB, cout), jnp.float32),
            pltpu.SemaphoreType.DMA,
        ],
    )
    def run(u_hbm, idx_hbm, stats_hbm, idx_v, rows_v, acc_v, sem):
        wid = lax.axis_index("s") * _NC + lax.axis_index("c")
        pbase = wid * _PTW
        pltpu.sync_copy(idx_hbm.at[pl.ds(pbase * _K, _IDXW)], idx_v)

        def group(g, carry):
            gbase = pbase + g * _GB
            for h in range(_GB // _GP):
                pltpu.async_copy(
                    u_hbm.at[idx_v.at[pl.ds((g * (_GB // _GP) + h)
                                            * (_GP * _K), _GP * _K)]],
                    rows_v, sem).wait()

                def point(p, c2):
                    def chunk(c, c3):
                        colo = c * _L
                        r0 = rows_v[p * _K, pl.ds(colo, _L)]
                        amax = r0
                        amin = r0
                        asum = r0
                        asq = r0 * r0
                        for j in range(1, _K):
                            r = rows_v[p * _K + j, pl.ds(colo, _L)]
                            amax = jnp.maximum(amax, r)
                            amin = jnp.minimum(amin, r)
                            asum = asum + r
                            asq = asq + r * r
                        acc_v[0, h * _GP + p, pl.ds(colo, _L)] = amax
                        acc_v[1, h * _GP + p, pl.ds(colo, _L)] = amin
                        acc_v[2, h * _GP + p, pl.ds(colo, _L)] = asum
                        acc_v[3, h * _GP + p, pl.ds(colo, _L)] = asq
                        return c3
                    lax.fori_loop(0, cout // _L, chunk, 0)
                    return c2
                lax.fori_loop(0, _GP, point, 0)
            for s in range(4):
                pltpu.sync_copy(acc_v.at[s],
                                stats_hbm.at[s].at[pl.ds(gbase, _GB)])
            return carry
        lax.fori_loop(0, _GRP, group, 0)

    return run


_RT = 16


def _reduce_call(cout, stats, w):
    rows = _BN // _RT

    def body(s_ref, q_ref, w_ref, out_ref):
        i = pl.program_id(0)

        @pl.when(i == 0)
        def _():
            out_ref[...] = jnp.zeros_like(out_ref)

        S = s_ref[0]
        Q = q_ref[0]
        W = w_ref[...]
        p1 = jnp.sum(S, axis=0, keepdims=True)
        p2 = jnp.sum(W, axis=0, keepdims=True)
        p3 = jnp.sum(Q, axis=0, keepdims=True)
        p4 = jnp.sum(S * W, axis=0, keepdims=True)
        p5 = jnp.sum(W * W, axis=0, keepdims=True)
        z = jnp.zeros((3, cout), jnp.float32)
        out_ref[...] += jnp.concatenate([p1, p2, p3, p4, p5, z], axis=0)

    return pl.pallas_call(
        body,
        grid=(_RT,),
        in_specs=[
            pl.BlockSpec((1, rows, cout), lambda i: (2, i, 0)),
            pl.BlockSpec((1, rows, cout), lambda i: (3, i, 0)),
            pl.BlockSpec((rows, cout), lambda i: (i, 0)),
        ],
        out_specs=pl.BlockSpec((8, cout), lambda i: (0, 0)),
        out_shape=jax.ShapeDtypeStruct((8, cout), jnp.float32),
    )(stats, stats, w)


def _bn_coeffs(sums, g, b, cnt):
    r1, r2, r3, r4, r5 = sums[0], sums[1], sums[2], sums[3], sums[4]
    mean = (r1 + _K * r2) / cnt
    ey2 = (r3 + 2.0 * r4 + _K * r5) / cnt
    var = ey2 - mean * mean
    scale = g / jnp.sqrt(var + 1e-5)
    shift = b - mean * scale
    return scale.reshape(1, -1), shift.reshape(1, -1)


def _cat_call(x1, x2, x3, st4, w4, sc4, sh4, w5T):
    def body(x1_ref, x2_ref, x3_ref, m_ref, mn_ref, wp_ref, sc_ref, sh_ref,
             w5_ref, ymax_ref, ymin_ref, sy_ref, sy2_ref):
        scale = sc_ref[...]
        shift = sh_ref[...]
        mt = jnp.where(scale >= 0.0, m_ref[0], mn_ref[0])
        y4 = scale * (mt + wp_ref[...]) + shift
        x4 = jnp.where(y4 > 0.0, y4, 0.2 * y4)
        cat = jnp.concatenate(
            [x1_ref[...][:, :64], x2_ref[...][:, :64], x3_ref[...], x4],
            axis=1)
        y = lax.dot_general(cat, w5_ref[...], (((1,), (0,)), ((), ())),
                            preferred_element_type=jnp.float32,
                            precision=lax.Precision.HIGHEST)
        ymax_ref[...] = jnp.max(y, axis=0, keepdims=True)[None]
        ymin_ref[...] = jnp.min(y, axis=0, keepdims=True)[None]
        sy_ref[...] = jnp.sum(y, axis=0, keepdims=True)[None]
        sy2_ref[...] = jnp.sum(y * y, axis=0, keepdims=True)[None]

    return pl.pallas_call(
        body,
        grid=(_B,),
        in_specs=[
            pl.BlockSpec((_N, 128), lambda b: (b, 0)),
            pl.BlockSpec((_N, 128), lambda b: (b, 0)),
            pl.BlockSpec((_N, 128), lambda b: (b, 0)),
            pl.BlockSpec((1, _N, 256), lambda b: (0, b, 0)),
            pl.BlockSpec((1, _N, 256), lambda b: (1, b, 0)),
            pl.BlockSpec((_N, 256), lambda b: (b, 0)),
            pl.BlockSpec((1, 256), lambda b: (0, 0)),
            pl.BlockSpec((1, 256), lambda b: (0, 0)),
            pl.BlockSpec((512, 1024), lambda b: (0, 0)),
        ],
        out_specs=[pl.BlockSpec((1, 1, 1024), lambda b: (b, 0, 0))] * 4,
        out_shape=[jax.ShapeDtypeStruct((_B, 1, 1024), jnp.float32)] * 4,
    )(x1, x2, x3, st4, st4, w4, sc4, sh4, w5T)


def _heads_call(ymax, ymin, sy, sy2, g5, b5, wmuT, bmu, wlvT, blv,
                wd1T, bd1, wd2T, bd2, wpT, bp, eps):
    def body(ymax_ref, ymin_ref, sy_ref, sy2_ref, g5_ref, b5_ref,
             wmu_ref, bmu_ref, wlv_ref, blv_ref, wd1_ref, bd1_ref,
             wd2_ref, bd2_ref, wp_ref, bp_ref, eps_ref,
             rec_ref, mu_ref, lv_ref, proj_ref):
        cnt = float(_BN)
        mean = jnp.sum(sy_ref[...], axis=0, keepdims=True) / cnt
        ey2 = jnp.sum(sy2_ref[...], axis=0, keepdims=True) / cnt
        var = ey2 - mean * mean
        scale = g5_ref[...] * lax.rsqrt(var + 1e-5)
        shift = b5_ref[...] - mean * scale
        yt = jnp.where(scale >= 0.0, ymax_ref[...], ymin_ref[...])
        feat = scale * yt + shift
        feat = jnp.where(feat > 0.0, feat, 0.2 * feat)

        def mm(a, wref, bref):
            return lax.dot_general(
                a, wref[...], (((1,), (0,)), ((), ())),
                preferred_element_type=jnp.float32,
                            precision=lax.Precision.HIGHEST) + bref[...]

        mu = mm(feat, wmu_ref, bmu_ref)
        logvar = mm(feat, wlv_ref, blv_ref)
        std = jnp.exp(0.5 * logvar)
        z = mu + eps_ref[...] * std
        h = mm(z, wd1_ref, bd1_ref)
        h = jnp.maximum(h, 0.0)
        rec = jnp.tanh(mm(h, wd2_ref, bd2_ref))
        rec_ref[...] = rec
        mu_ref[...] = mu
        lv_ref[...] = logvar
        proj_ref[...] = mm(feat, wp_ref, bp_ref)

    return pl.pallas_call(
        body,
        out_shape=[
            jax.ShapeDtypeStruct((_B, 3072), jnp.float32),
            jax.ShapeDtypeStruct((_B, 128), jnp.float32),
            jax.ShapeDtypeStruct((_B, 128), jnp.float32),
            jax.ShapeDtypeStruct((_B, 128), jnp.float32),
        ],
    )(ymax, ymin, sy, sy2, g5, b5, wmuT, bmu, wlvT, blv,
      wd1T, bd1, wd2T, bd2, wpT, bp, eps)


def _edge_weights(W, cin, rpad=0, cpad=0):
    Wa = W[:, :cin]
    Wb = W[:, cin:]
    waT = Wa.T
    wdT = (Wb - Wa).T
    if rpad or cpad:
        waT = jnp.pad(waT, ((0, rpad), (0, cpad)))
        wdT = jnp.pad(wdT, ((0, rpad), (0, cpad)))
    return waT, wdT


def kernel(x, W1, g1, b1, W2, g2, b2, W3, g3, b3, W4, g4, b4, W5, g5, b5,
           Wmu, bmu, Wlv, blv, Wd1, bd1, Wd2, bd2, Wp, bp, eps):
    cnt_e = float(_BN * _K)

    xt0 = jnp.transpose(x, (0, 2, 1)).reshape(_BN, 3)
    xt0 = jnp.pad(xt0, ((0, 0), (0, 5)))
    g1p = jnp.pad(g1, (0, 64))
    b1p = jnp.pad(b1, (0, 64))
    waT1, wdT1 = _edge_weights(W1, 3, rpad=5, cpad=64)
    idx1, u1, w1 = _layer_call(8, 128, True, (xt0, waT1, wdT1))
    st1 = _sc_gather(128)(u1, idx1[:, :_K].reshape(-1))
    sc1, sh1 = _bn_coeffs(_reduce_call(128, st1, w1), g1p, b1p, cnt_e)

    g2p = jnp.pad(g2, (0, 64))
    b2p = jnp.pad(b2, (0, 64))
    waT2, wdT2 = _edge_weights(W2, 64, rpad=64, cpad=64)
    x1o, idx2, u2, w2 = _layer_call(
        128, 128, False, (st1, st1, w1, sc1, sh1, waT2, wdT2))
    st2 = _sc_gather(128)(u2, idx2[:, :_K].reshape(-1))
    sc2, sh2 = _bn_coeffs(_reduce_call(128, st2, w2), g2p, b2p, cnt_e)

    waT3, wdT3 = _edge_weights(W3, 64, rpad=64, cpad=0)
    x2o, idx3, u3, w3 = _layer_call(
        128, 128, False, (st2, st2, w2, sc2, sh2, waT3, wdT3))
    st3 = _sc_gather(128)(u3, idx3[:, :_K].reshape(-1))
    sc3, sh3 = _bn_coeffs(_reduce_call(128, st3, w3), g3, b3, cnt_e)

    waT4, wdT4 = _edge_weights(W4, 128)
    x3o, idx4, u4, w4 = _layer_call(
        128, 256, False, (st3, st3, w3, sc3, sh3, waT4, wdT4))
    st4 = _sc_gather(256)(u4, idx4[:, :_K].reshape(-1))
    sc4, sh4 = _bn_coeffs(_reduce_call(256, st4, w4), g4, b4, cnt_e)

    ymax, ymin, sy, sy2 = _cat_call(x1o, x2o, x3o, st4, w4, sc4, sh4, W5.T)

    rec, mu, logvar, proj = _heads_call(
        ymax.reshape(_B, 1024), ymin.reshape(_B, 1024),
        sy.reshape(_B, 1024), sy2.reshape(_B, 1024),
        g5.reshape(1, -1), b5.reshape(1, -1),
        Wmu.T, bmu.reshape(1, -1), Wlv.T, blv.reshape(1, -1),
        Wd1.T, bd1.reshape(1, -1), Wd2.T, bd2.reshape(1, -1),
        Wp.T, bp.reshape(1, -1), eps)
    return (rec.reshape(-1, 3, 1024), mu, logvar, proj)


# speedup vs baseline: 11.2427x; 1.0715x over previous
import functools

import jax
import jax.numpy as jnp
from jax import lax
from jax.experimental import pallas as pl
from jax.experimental.pallas import tpu as pltpu
from jax.experimental.pallas import tpu_sc as plsc

_B, _N, _K = 8, 1024, 20
_BN = _B * _N
_KPAD = 32
_NEG = -3.0e38

_NC, _NS, _L = 2, 16, 16
_NW = _NC * _NS
_PTW = _BN // _NW
_GP = 4
_GB = 8
_GRP = _PTW // _GB
_IDXW = _PTW * _K


def _layer_call(cin, cout, first, args):
    def body(*refs):
        if first:
            (xt_ref, waT_ref, wdT_ref, idx_ref, u_ref, w_ref) = refs
            xt = xt_ref[...]
        else:
            (m_ref, mn_ref, wp_ref, sc_ref, sh_ref, waT_ref, wdT_ref,
             xt_out_ref, idx_ref, u_ref, w_ref) = refs
            scale = sc_ref[...]
            shift = sh_ref[...]
            mt = jnp.where(scale >= 0.0, m_ref[0], mn_ref[0])
            y = scale * (mt + wp_ref[...]) + shift
            xt = jnp.where(y > 0.0, y, 0.2 * y)
            xt_out_ref[...] = xt
        b = pl.program_id(0)
        xb = xt.astype(jnp.bfloat16)
        G = lax.dot_general(xb, xb, (((1,), (1,)), ((), ())),
                            preferred_element_type=jnp.float32)
        inner = -2.0 * G
        xsq = xt * xt
        xxcol = jnp.sum(xsq, axis=1, keepdims=True)
        ones_row = jnp.ones((1, cin), jnp.float32)
        xxrow = lax.dot_general(ones_row, xsq, (((1,), (1,)), ((), ())),
                                preferred_element_type=jnp.float32,
                                precision=lax.Precision.HIGHEST)
        pair = ((0.0 - xxcol) - inner) - xxrow
        col = lax.broadcasted_iota(jnp.int32, (_N, _N), 1)
        cols = []
        for k in range(_K):
            m = jnp.max(pair, axis=1, keepdims=True)
            cand = jnp.where(pair == m, col, _N)
            ik = jnp.min(cand, axis=1, keepdims=True)
            cols.append(ik)
            if k + 1 < _K:
                pair = jnp.where(col == ik, _NEG, pair)
        idxmat = jnp.concatenate(cols + [cols[-1]] * (_KPAD - _K), axis=1)
        idx_ref[...] = idxmat + b * _N
        u_ref[...] = lax.dot_general(xt, waT_ref[...], (((1,), (0,)), ((), ())),
                                     preferred_element_type=jnp.float32,
                            precision=lax.Precision.HIGHEST)
        w_ref[...] = lax.dot_general(xt, wdT_ref[...], (((1,), (0,)), ((), ())),
                                     preferred_element_type=jnp.float32,
                            precision=lax.Precision.HIGHEST)

    wspec = [
        pl.BlockSpec((cin, cout), lambda b: (0, 0)),
        pl.BlockSpec((cin, cout), lambda b: (0, 0)),
    ]
    if first:
        in_specs = [pl.BlockSpec((_N, cin), lambda b: (b, 0))] + wspec
        out_shapes = []
        out_specs = []
    else:
        in_specs = [
            pl.BlockSpec((1, _N, cin), lambda b: (0, b, 0)),
            pl.BlockSpec((1, _N, cin), lambda b: (1, b, 0)),
            pl.BlockSpec((_N, cin), lambda b: (b, 0)),
            pl.BlockSpec((1, cin), lambda b: (0, 0)),
            pl.BlockSpec((1, cin), lambda b: (0, 0)),
        ] + wspec
        out_shapes = [jax.ShapeDtypeStruct((_BN, cin), jnp.float32)]
        out_specs = [pl.BlockSpec((_N, cin), lambda b: (b, 0))]
    out_shapes += [
        jax.ShapeDtypeStruct((_BN, _KPAD), jnp.int32),
        jax.ShapeDtypeStruct((_BN, cout), jnp.float32),
        jax.ShapeDtypeStruct((_BN, cout), jnp.float32),
    ]
    out_specs += [
        pl.BlockSpec((_N, _KPAD), lambda b: (b, 0)),
        pl.BlockSpec((_N, cout), lambda b: (b, 0)),
        pl.BlockSpec((_N, cout), lambda b: (b, 0)),
    ]
    return pl.pallas_call(
        body,
        grid=(_B,),
        in_specs=in_specs,
        out_specs=out_specs,
        out_shape=out_shapes,
    )(*args)


def _sc_gather(cout):
    mesh = plsc.VectorSubcoreMesh(core_axis_name="c", subcore_axis_name="s",
                                  num_cores=_NC, num_subcores=_NS)

    @functools.partial(
        pl.kernel,
        out_type=jax.ShapeDtypeStruct((4, _BN, cout), jnp.float32),
        mesh=mesh,
        scratch_types=[
            pltpu.VMEM((_IDXW,), jnp.int32),
            pltpu.VMEM((2, _GP * _K, cout), jnp.float32),
            pltpu.VMEM((4, _GB, cout), jnp.float32),
            pltpu.SemaphoreType.DMA,
            pltpu.SemaphoreType.DMA,
        ],
    )
    def run(u_hbm, idx_hbm, stats_hbm, idx_v, rows_v, acc_v, sem0, sem1):
        wid = lax.axis_index("s") * _NC + lax.axis_index("c")
        pbase = wid * _PTW
        pltpu.sync_copy(idx_hbm.at[pl.ds(pbase * _K, _IDXW)], idx_v)
        sems = (sem0, sem1)

        def group(g, carry):
            gbase = pbase + g * _GB
            descs = [
                pltpu.async_copy(
                    u_hbm.at[idx_v.at[pl.ds((g * (_GB // _GP) + h)
                                            * (_GP * _K), _GP * _K)]],
                    rows_v.at[h], sems[h])
                for h in range(_GB // _GP)
            ]
            for h in range(_GB // _GP):
                descs[h].wait()

                def point(p, c2):
                    def chunk(c, c3):
                        colo = c * _L
                        r0 = rows_v[h, p * _K, pl.ds(colo, _L)]
                        amax = r0
                        amin = r0
                        asum = r0
                        asq = r0 * r0
                        for j in range(1, _K):
                            r = rows_v[h, p * _K + j, pl.ds(colo, _L)]
                            amax = jnp.maximum(amax, r)
                            amin = jnp.minimum(amin, r)
                            asum = asum + r
                            asq = asq + r * r
                        acc_v[0, h * _GP + p, pl.ds(colo, _L)] = amax
                        acc_v[1, h * _GP + p, pl.ds(colo, _L)] = amin
                        acc_v[2, h * _GP + p, pl.ds(colo, _L)] = asum
                        acc_v[3, h * _GP + p, pl.ds(colo, _L)] = asq
                        return c3
                    lax.fori_loop(0, cout // _L, chunk, 0)
                    return c2
                lax.fori_loop(0, _GP, point, 0)
            for s in range(4):
                pltpu.sync_copy(acc_v.at[s],
                                stats_hbm.at[s].at[pl.ds(gbase, _GB)])
            return carry
        lax.fori_loop(0, _GRP, group, 0)

    return run


_RT = 16


def _reduce_call(cout, stats, w):
    rows = _BN // _RT

    def body(s_ref, q_ref, w_ref, out_ref):
        i = pl.program_id(0)

        @pl.when(i == 0)
        def _():
            out_ref[...] = jnp.zeros_like(out_ref)

        S = s_ref[0]
        Q = q_ref[0]
        W = w_ref[...]
        p1 = jnp.sum(S, axis=0, keepdims=True)
        p2 = jnp.sum(W, axis=0, keepdims=True)
        p3 = jnp.sum(Q, axis=0, keepdims=True)
        p4 = jnp.sum(S * W, axis=0, keepdims=True)
        p5 = jnp.sum(W * W, axis=0, keepdims=True)
        z = jnp.zeros((3, cout), jnp.float32)
        out_ref[...] += jnp.concatenate([p1, p2, p3, p4, p5, z], axis=0)

    return pl.pallas_call(
        body,
        grid=(_RT,),
        in_specs=[
            pl.BlockSpec((1, rows, cout), lambda i: (2, i, 0)),
            pl.BlockSpec((1, rows, cout), lambda i: (3, i, 0)),
            pl.BlockSpec((rows, cout), lambda i: (i, 0)),
        ],
        out_specs=pl.BlockSpec((8, cout), lambda i: (0, 0)),
        out_shape=jax.ShapeDtypeStruct((8, cout), jnp.float32),
    )(stats, stats, w)


def _bn_coeffs(sums, g, b, cnt):
    r1, r2, r3, r4, r5 = sums[0], sums[1], sums[2], sums[3], sums[4]
    mean = (r1 + _K * r2) / cnt
    ey2 = (r3 + 2.0 * r4 + _K * r5) / cnt
    var = ey2 - mean * mean
    scale = g / jnp.sqrt(var + 1e-5)
    shift = b - mean * scale
    return scale.reshape(1, -1), shift.reshape(1, -1)


def _cat_call(x1, x2, x3, st4, w4, sc4, sh4, w5T):
    def body(x1_ref, x2_ref, x3_ref, m_ref, mn_ref, wp_ref, sc_ref, sh_ref,
             w5_ref, ymax_ref, ymin_ref, sy_ref, sy2_ref):
        scale = sc_ref[...]
        shift = sh_ref[...]
        mt = jnp.where(scale >= 0.0, m_ref[0], mn_ref[0])
        y4 = scale * (mt + wp_ref[...]) + shift
        x4 = jnp.where(y4 > 0.0, y4, 0.2 * y4)
        cat = jnp.concatenate(
            [x1_ref[...][:, :64], x2_ref[...][:, :64], x3_ref[...], x4],
            axis=1)
        y = lax.dot_general(cat, w5_ref[...], (((1,), (0,)), ((), ())),
                            preferred_element_type=jnp.float32,
                            precision=lax.Precision.HIGHEST)
        ymax_ref[...] = jnp.max(y, axis=0, keepdims=True)[None]
        ymin_ref[...] = jnp.min(y, axis=0, keepdims=True)[None]
        sy_ref[...] = jnp.sum(y, axis=0, keepdims=True)[None]
        sy2_ref[...] = jnp.sum(y * y, axis=0, keepdims=True)[None]

    return pl.pallas_call(
        body,
        grid=(_B,),
        in_specs=[
            pl.BlockSpec((_N, 128), lambda b: (b, 0)),
            pl.BlockSpec((_N, 128), lambda b: (b, 0)),
            pl.BlockSpec((_N, 128), lambda b: (b, 0)),
            pl.BlockSpec((1, _N, 256), lambda b: (0, b, 0)),
            pl.BlockSpec((1, _N, 256), lambda b: (1, b, 0)),
            pl.BlockSpec((_N, 256), lambda b: (b, 0)),
            pl.BlockSpec((1, 256), lambda b: (0, 0)),
            pl.BlockSpec((1, 256), lambda b: (0, 0)),
            pl.BlockSpec((512, 1024), lambda b: (0, 0)),
        ],
        out_specs=[pl.BlockSpec((1, 1, 1024), lambda b: (b, 0, 0))] * 4,
        out_shape=[jax.ShapeDtypeStruct((_B, 1, 1024), jnp.float32)] * 4,
    )(x1, x2, x3, st4, st4, w4, sc4, sh4, w5T)


def _heads_call(ymax, ymin, sy, sy2, g5, b5, wmuT, bmu, wlvT, blv,
                wd1T, bd1, wd2T, bd2, wpT, bp, eps):
    def body(ymax_ref, ymin_ref, sy_ref, sy2_ref, g5_ref, b5_ref,
             wmu_ref, bmu_ref, wlv_ref, blv_ref, wd1_ref, bd1_ref,
             wd2_ref, bd2_ref, wp_ref, bp_ref, eps_ref,
             rec_ref, mu_ref, lv_ref, proj_ref):
        cnt = float(_BN)
        mean = jnp.sum(sy_ref[...], axis=0, keepdims=True) / cnt
        ey2 = jnp.sum(sy2_ref[...], axis=0, keepdims=True) / cnt
        var = ey2 - mean * mean
        scale = g5_ref[...] * lax.rsqrt(var + 1e-5)
        shift = b5_ref[...] - mean * scale
        yt = jnp.where(scale >= 0.0, ymax_ref[...], ymin_ref[...])
        feat = scale * yt + shift
        feat = jnp.where(feat > 0.0, feat, 0.2 * feat)

        def mm(a, wref, bref):
            return lax.dot_general(
                a, wref[...], (((1,), (0,)), ((), ())),
                preferred_element_type=jnp.float32,
                            precision=lax.Precision.HIGHEST) + bref[...]

        mu = mm(feat, wmu_ref, bmu_ref)
        logvar = mm(feat, wlv_ref, blv_ref)
        std = jnp.exp(0.5 * logvar)
        z = mu + eps_ref[...] * std
        h = mm(z, wd1_ref, bd1_ref)
        h = jnp.maximum(h, 0.0)
        rec = jnp.tanh(mm(h, wd2_ref, bd2_ref))
        rec_ref[...] = rec
        mu_ref[...] = mu
        lv_ref[...] = logvar
        proj_ref[...] = mm(feat, wp_ref, bp_ref)

    return pl.pallas_call(
        body,
        out_shape=[
            jax.ShapeDtypeStruct((_B, 3072), jnp.float32),
            jax.ShapeDtypeStruct((_B, 128), jnp.float32),
            jax.ShapeDtypeStruct((_B, 128), jnp.float32),
            jax.ShapeDtypeStruct((_B, 128), jnp.float32),
        ],
    )(ymax, ymin, sy, sy2, g5, b5, wmuT, bmu, wlvT, blv,
      wd1T, bd1, wd2T, bd2, wpT, bp, eps)


def _edge_weights(W, cin, rpad=0, cpad=0):
    Wa = W[:, :cin]
    Wb = W[:, cin:]
    waT = Wa.T
    wdT = (Wb - Wa).T
    if rpad or cpad:
        waT = jnp.pad(waT, ((0, rpad), (0, cpad)))
        wdT = jnp.pad(wdT, ((0, rpad), (0, cpad)))
    return waT, wdT


def kernel(x, W1, g1, b1, W2, g2, b2, W3, g3, b3, W4, g4, b4, W5, g5, b5,
           Wmu, bmu, Wlv, blv, Wd1, bd1, Wd2, bd2, Wp, bp, eps):
    cnt_e = float(_BN * _K)

    xt0 = jnp.transpose(x, (0, 2, 1)).reshape(_BN, 3)
    xt0 = jnp.pad(xt0, ((0, 0), (0, 5)))
    g1p = jnp.pad(g1, (0, 64))
    b1p = jnp.pad(b1, (0, 64))
    waT1, wdT1 = _edge_weights(W1, 3, rpad=5, cpad=64)
    idx1, u1, w1 = _layer_call(8, 128, True, (xt0, waT1, wdT1))
    st1 = _sc_gather(128)(u1, idx1[:, :_K].reshape(-1))
    sc1, sh1 = _bn_coeffs(_reduce_call(128, st1, w1), g1p, b1p, cnt_e)

    g2p = jnp.pad(g2, (0, 64))
    b2p = jnp.pad(b2, (0, 64))
    waT2, wdT2 = _edge_weights(W2, 64, rpad=64, cpad=64)
    x1o, idx2, u2, w2 = _layer_call(
        128, 128, False, (st1, st1, w1, sc1, sh1, waT2, wdT2))
    st2 = _sc_gather(128)(u2, idx2[:, :_K].reshape(-1))
    sc2, sh2 = _bn_coeffs(_reduce_call(128, st2, w2), g2p, b2p, cnt_e)

    waT3, wdT3 = _edge_weights(W3, 64, rpad=64, cpad=0)
    x2o, idx3, u3, w3 = _layer_call(
        128, 128, False, (st2, st2, w2, sc2, sh2, waT3, wdT3))
    st3 = _sc_gather(128)(u3, idx3[:, :_K].reshape(-1))
    sc3, sh3 = _bn_coeffs(_reduce_call(128, st3, w3), g3, b3, cnt_e)

    waT4, wdT4 = _edge_weights(W4, 128)
    x3o, idx4, u4, w4 = _layer_call(
        128, 256, False, (st3, st3, w3, sc3, sh3, waT4, wdT4))
    st4 = _sc_gather(256)(u4, idx4[:, :_K].reshape(-1))
    sc4, sh4 = _bn_coeffs(_reduce_call(256, st4, w4), g4, b4, cnt_e)

    ymax, ymin, sy, sy2 = _cat_call(x1o, x2o, x3o, st4, w4, sc4, sh4, W5.T)

    rec, mu, logvar, proj = _heads_call(
        ymax.reshape(_B, 1024), ymin.reshape(_B, 1024),
        sy.reshape(_B, 1024), sy2.reshape(_B, 1024),
        g5.reshape(1, -1), b5.reshape(1, -1),
        Wmu.T, bmu.reshape(1, -1), Wlv.T, blv.reshape(1, -1),
        Wd1.T, bd1.reshape(1, -1), Wd2.T, bd2.reshape(1, -1),
        Wp.T, bp.reshape(1, -1), eps)
    return (rec.reshape(-1, 3, 1024), mu, logvar, proj)
